# baseline (speedup 1.0000x reference)
import jax
import jax.numpy as jnp
from jax.experimental import pallas as pl

Q = 1024
D = 16
SUB = 3584
L = 128


def _knn_kernel(xa_ref, yt_ref, o_ref):
    xa = xa_ref[...]
    yt = yt_ref[...]
    kp = yt.shape[1]
    ytf = yt.astype(jnp.float32)
    y2 = jnp.sum(ytf * ytf, axis=0, keepdims=True)
    ya = jnp.concatenate([yt, y2.astype(jnp.bfloat16)], axis=0)
    bm = None
    for s in range(kp // SUB):
        t = jax.lax.dot_general(
            xa, ya[:, s * SUB:(s + 1) * SUB],
            dimension_numbers=(((1,), (0,)), ((), ())),
            preferred_element_type=jnp.float32,
        )
        for j in range(SUB // L):
            c = t[:, j * L:(j + 1) * L]
            bm = c if bm is None else jnp.minimum(bm, c)
    xm2 = xa[:, :D].astype(jnp.float32)
    x2 = jnp.sum(xm2 * xm2, axis=1) * 0.25
    d2 = jnp.maximum(jnp.min(bm, axis=1) + x2, 0.0)
    o_ref[...] = jnp.sqrt(d2) * 10.0


@jax.jit
def kernel(mutation_dist, train_data):
    k = train_data.shape[0]
    kp = ((k + SUB - 1) // SUB) * SUB
    yt = jnp.pad(train_data.T.astype(jnp.bfloat16), ((0, 0), (0, kp - k)),
                 constant_values=1e4)
    xb = mutation_dist.astype(jnp.bfloat16)
    xa = jnp.concatenate(
        [xb * jnp.bfloat16(-2.0),
         jnp.ones((Q, 1), jnp.bfloat16)], axis=1)
    return pl.pallas_call(
        _knn_kernel,
        in_specs=[
            pl.BlockSpec((Q, D + 1), lambda: (0, 0)),
            pl.BlockSpec((D, kp), lambda: (0, 0)),
        ],
        out_specs=pl.BlockSpec((Q,), lambda: (0,)),
        out_shape=jax.ShapeDtypeStruct((Q,), jnp.float32),
    )(xa, yt)


# speedup vs baseline: 1.0364x; 1.0364x over previous
import functools

import jax
import jax.numpy as jnp
from jax.experimental import pallas as pl
from jax.experimental.pallas import tpu as pltpu

Q = 1024
D = 16
SUB = 3584
L = 128
NSTEPS = 2


def _knn_kernel(xa_ref, yt_ref, o_ref, acc_ref, *, nsteps):
    i = pl.program_id(0)
    xa = xa_ref[...]
    yt = yt_ref[...]
    chunk = yt.shape[1]
    ytf = yt.astype(jnp.float32)
    y2 = jnp.sum(ytf * ytf, axis=0, keepdims=True)
    ya = jnp.concatenate([yt, y2.astype(jnp.bfloat16)], axis=0)
    bm = None
    for s in range(chunk // SUB):
        t = jax.lax.dot_general(
            xa, ya[:, s * SUB:(s + 1) * SUB],
            dimension_numbers=(((1,), (0,)), ((), ())),
            preferred_element_type=jnp.float32,
        )
        for j in range(SUB // L):
            c = t[:, j * L:(j + 1) * L]
            bm = c if bm is None else jnp.minimum(bm, c)

    @pl.when(i == 0)
    def _init():
        acc_ref[...] = bm

    @pl.when(i > 0)
    def _update():
        acc_ref[...] = jnp.minimum(acc_ref[...], bm)

    @pl.when(i == nsteps - 1)
    def _finalize():
        xm2 = xa[:, :D].astype(jnp.float32)
        x2 = jnp.sum(xm2 * xm2, axis=1) * 0.25
        d2 = jnp.maximum(jnp.min(acc_ref[...], axis=1) + x2, 0.0)
        o_ref[...] = jnp.sqrt(d2) * 10.0


@jax.jit
def kernel(mutation_dist, train_data):
    k = train_data.shape[0]
    step = NSTEPS * SUB
    kp = ((k + step - 1) // step) * step
    chunk = kp // NSTEPS
    yt = jnp.pad(train_data.T.astype(jnp.bfloat16), ((0, 0), (0, kp - k)),
                 constant_values=1e4)
    xb = mutation_dist.astype(jnp.bfloat16)
    xa = jnp.concatenate(
        [xb * jnp.bfloat16(-2.0),
         jnp.ones((Q, 1), jnp.bfloat16)], axis=1)
    return pl.pallas_call(
        functools.partial(_knn_kernel, nsteps=NSTEPS),
        grid=(NSTEPS,),
        in_specs=[
            pl.BlockSpec((Q, D + 1), lambda i: (0, 0)),
            pl.BlockSpec((D, chunk), lambda i: (0, i)),
        ],
        out_specs=pl.BlockSpec((Q,), lambda i: (0,)),
        out_shape=jax.ShapeDtypeStruct((Q,), jnp.float32),
        scratch_shapes=[pltpu.VMEM((Q, L), jnp.float32)],
        compiler_params=pltpu.CompilerParams(
            dimension_semantics=("arbitrary",),
        ),
    )(xa, yt)
